# baseline (speedup 1.0000x reference)
import functools

import jax
import jax.numpy as jnp
from jax import lax
from jax.experimental import pallas as pl
from jax.experimental.pallas import tpu as pltpu
from jax.experimental.pallas import tpu_sc as plsc

_L = 16


def _make_gather(nw, ch, cw, lpb, t):
    per_w = ch * cw
    mesh = plsc.VectorSubcoreMesh(core_axis_name="c", subcore_axis_name="s")
    nc = mesh.num_cores

    mul, sh = 5243, 18
    assert lpb == 50 and per_w <= 6400
    ow = cw // 2

    @functools.partial(
        pl.kernel,
        out_type=jax.ShapeDtypeStruct((nw * ch * ow, 128), jnp.float32),
        mesh=mesh,
        scratch_types=[
            pltpu.VMEM((ch, cw), jnp.int32),
            pltpu.VMEM((ch, cw), jnp.int32),
            pltpu.VMEM((2, cw, 128), jnp.float32),
            pltpu.VMEM((2, ow, 128), jnp.float32),
            pltpu.SMEM((cw,), jnp.int32),
            pltpu.SemaphoreType.DMA,
            pltpu.SemaphoreType.DMA,
        ],
        compiler_params=pltpu.CompilerParams(
            use_tc_tiling_on_sc=True, needs_layout_passes=False),
    )
    def k(bounds_hbm, table_hbm, out_hbm, idx_v, par_v, rows_v, pack_v,
          par_s, gsem, osem):
        wid = lax.axis_index("s") * nc + lax.axis_index("c")
        pltpu.sync_copy(bounds_hbm.at[wid], idx_v)
        wbase = wid * (per_w // lpb)

        lane = lax.iota(jnp.int32, 16)

        def compute_idx(j, carry):
            for kk in range(cw // _L):
                n = j * cw + kk * _L + lane
                b = lax.shift_right_logical(n * mul, sh)
                raw = idx_v[j, pl.ds(kk * _L, _L)]
                r = lax.shift_right_logical(raw, 2) + (wbase + b) * t
                idx_v[j, pl.ds(kk * _L, _L)] = lax.shift_right_logical(r, 1)
                par_v[j, pl.ds(kk * _L, _L)] = lax.bitwise_and(r, 1) * 64
            return carry

        lax.fori_loop(0, ch, compute_idx, 0)

        def gather(j):
            return pltpu.async_copy(
                table_hbm.at[idx_v.at[j]], rows_v.at[lax.rem(j, 2)], gsem)

        gather(0)
        gather(1)

        def step(j, carry):
            buf = lax.rem(j, 2)
            pltpu.make_async_copy(
                table_hbm.at[idx_v.at[j]], rows_v.at[buf], gsem).wait()

            @pl.when(j >= 2)
            def _():
                pltpu.make_async_copy(
                    pack_v.at[buf],
                    out_hbm.at[pl.ds(0, ow)], osem).wait()

            rows_b = rows_v.at[buf]
            pack_b = pack_v.at[buf]
            for g in range(cw // _L):
                o_vec = g * _L + lane
                p64 = par_v[j, pl.ds(g * _L, _L)]
                oo = lax.shift_right_logical(o_vec, 1)
                hb = lax.bitwise_and(o_vec, 1) * 64
                for c in range(64):
                    x = plsc.load_gather(rows_b, [o_vec, p64 + c])
                    plsc.store_scatter(pack_b, [oo, hb + c], x)

            @pl.when(j + 2 < ch)
            def _():
                gather(j + 2)

            pltpu.async_copy(
                pack_v.at[buf],
                out_hbm.at[pl.ds((wid * ch + j) * ow, ow)], osem)
            return carry

        lax.fori_loop(0, ch, step, 0)

        for _ in range(2):
            pltpu.make_async_copy(
                pack_v.at[0], out_hbm.at[pl.ds(0, ow)], osem).wait()

    return k


def kernel(X, bounds):
    B, T, D = X.shape
    Bb, L = bounds.shape
    NW = 32
    assert B == Bb and (B * L) % NW == 0 and (T * D) % 128 == 0
    per_w = B * L // NW
    CW = 128
    CH = per_w // CW
    table = X.reshape(B * T * D // 128, 128)
    b3 = bounds.astype(jnp.int32).reshape(NW, CH, CW)
    fn = _make_gather(NW, CH, CW, L, T)
    out = fn(b3, table)
    return out.reshape(B, L, D)


# speedup vs baseline: 1.4751x; 1.4751x over previous
import functools

import jax
import jax.numpy as jnp
from jax import lax
from jax.experimental import pallas as pl
from jax.experimental.pallas import tpu as pltpu
from jax.experimental.pallas import tpu_sc as plsc

_NB = 2
_CW = 128


def _make_gather(b, t, d, lpb, nw):
    bpw = b // nw
    mesh = plsc.VectorSubcoreMesh(core_axis_name="c", subcore_axis_name="s")
    nc = mesh.num_cores
    assert bpw % _NB == 0

    @functools.partial(
        pl.kernel,
        out_type=jax.ShapeDtypeStruct((b * lpb, d), jnp.float32),
        mesh=mesh,
        scratch_types=[
            pltpu.VMEM((bpw, _CW), jnp.int32),
            pltpu.VMEM_SHARED((16, _NB, t, d), jnp.float32),
            pltpu.VMEM((_NB, _CW, d), jnp.float32),
            pltpu.SemaphoreType.DMA,
            pltpu.SemaphoreType.DMA,
            pltpu.SemaphoreType.DMA,
        ],
        compiler_params=pltpu.CompilerParams(use_tc_tiling_on_sc=False),
    )
    def k(idx_hbm, x_hbm, out_hbm, idx_v, slab_sh, rows_v, ssem, gsem, osem):
        sid = lax.axis_index("s")
        wid = sid * nc + lax.axis_index("c")
        wbase = wid * bpw
        slab_v = slab_sh.at[sid]
        pltpu.sync_copy(idx_hbm.at[pl.ds(wbase, bpw)], idx_v)

        def slab_dma(j, s):
            return pltpu.async_copy(x_hbm.at[wbase + j], slab_v.at[s], ssem)

        for s in range(_NB):
            slab_dma(s, s)

        def step(i, carry):
            j0 = i * _NB
            for s in range(_NB):
                j = j0 + s
                pltpu.make_async_copy(
                    x_hbm.at[wbase + j], slab_v.at[s], ssem).wait()
                @pl.when(i > 0)
                def _():
                    pltpu.make_async_copy(
                        rows_v.at[s].at[pl.ds(0, lpb)],
                        out_hbm.at[pl.ds((wbase + j - _NB) * lpb, lpb)],
                        osem).wait()
                pltpu.async_copy(
                    slab_v.at[s].at[idx_v.at[j]], rows_v.at[s], gsem).wait()
                @pl.when(j + _NB < bpw)
                def _():
                    slab_dma(j + _NB, s)
                pltpu.async_copy(
                    rows_v.at[s].at[pl.ds(0, lpb)],
                    out_hbm.at[pl.ds((wbase + j) * lpb, lpb)],
                    osem)
            return carry

        lax.fori_loop(0, bpw // _NB, step, 0)

        for s in range(_NB):
            pltpu.make_async_copy(
                rows_v.at[s].at[pl.ds(0, lpb)],
                out_hbm.at[pl.ds((wbase + bpw - _NB + s) * lpb, lpb)],
                osem).wait()

    return k


def kernel(X, bounds):
    B, T, D = X.shape
    Bb, L = bounds.shape
    NW = 32
    assert B == Bb and B % NW == 0
    idx = (bounds // 4).astype(jnp.int32)
    idx = jnp.pad(idx, ((0, 0), (0, _CW - L)))
    fn = _make_gather(B, T, D, L, NW)
    out = fn(idx, X)
    return out.reshape(B, L, D)
